# baseline (speedup 1.0000x reference)
import functools

import jax
import jax.numpy as jnp
from jax import lax
from jax.experimental import pallas as pl
from jax.experimental.pallas import tpu as pltpu
from jax.experimental.pallas import tpu_sc as plsc


def _make_sc_gather(V, D, B):
    info = plsc.get_sparse_core_info()
    nw = info.num_cores * info.num_subcores
    b_per_w = B // nw
    mesh = plsc.VectorSubcoreMesh(core_axis_name="c", subcore_axis_name="s")

    @functools.partial(
        pl.kernel,
        mesh=mesh,
        out_type=[
            jax.ShapeDtypeStruct((B, D), jnp.float32),
            jax.ShapeDtypeStruct((B, D), jnp.float32),
        ],
        scratch_types=[
            pltpu.VMEM((b_per_w,), jnp.int32),
            pltpu.VMEM((b_per_w, D), jnp.float32),
            pltpu.SemaphoreType.DMA,
        ],
    )
    def gather_k(emb_hbm, idx1_hbm, idx2_hbm, out1_hbm, out2_hbm,
                 idx_v, rows_v, sem):
        wid = lax.axis_index("s") * info.num_cores + lax.axis_index("c")
        base = wid * b_per_w
        pltpu.sync_copy(idx1_hbm.at[pl.ds(base, b_per_w)], idx_v)
        pltpu.async_copy(emb_hbm.at[idx_v], rows_v, sem).wait()
        pltpu.sync_copy(rows_v, out1_hbm.at[pl.ds(base, b_per_w)])
        pltpu.sync_copy(idx2_hbm.at[pl.ds(base, b_per_w)], idx_v)
        pltpu.async_copy(emb_hbm.at[idx_v], rows_v, sem).wait()
        pltpu.sync_copy(rows_v, out2_hbm.at[pl.ds(base, b_per_w)])

    return gather_k


_HC = 250


_SUB = 32


def _mlp_body(x1_ref, x2_ref, w1_ref, b1_ref, w2_ref, b2_ref, o_ref):
    d = x1_ref.shape[1]
    bm = x1_ref.shape[0]
    sb = bm // _SUB
    w1 = w1_ref[...]
    w2 = w2_ref[...]
    b1 = b1_ref[...]
    b2 = b2_ref[...]
    for j in range(_SUB):
        r = pl.ds(j * sb, sb)
        x = jnp.concatenate([x1_ref[r, :].astype(jnp.bfloat16),
                             x2_ref[r, :].astype(jnp.bfloat16)], axis=1)
        h = jnp.dot(x, w1, preferred_element_type=jnp.float32)
        h = jnp.maximum((h + b1).astype(jnp.bfloat16),
                        jnp.bfloat16(0.0))
        o = jnp.dot(h, w2, preferred_element_type=jnp.float32)
        e = jnp.exp(o + b2)
        y = e / jnp.sum(e, axis=1, keepdims=True)
        o_ref[:, r] = y.T


_NCHUNK = 1
_BM = 16384


def kernel(in1, in2, emb, W1, b1, W2, b2):
    B = in1.shape[0]
    V, D = emb.shape
    H = W1.shape[1]
    O = W2.shape[1]

    in1 = in1.astype(jnp.int32)
    in2 = in2.astype(jnp.int32)

    cb = B // _NCHUNK
    gather = _make_sc_gather(V, D, cb)

    mlp = pl.pallas_call(
        _mlp_body,
        grid=(cb // _BM,),
        in_specs=[
            pl.BlockSpec((_BM, D), lambda i: (i, 0)),
            pl.BlockSpec((_BM, D), lambda i: (i, 0)),
            pl.BlockSpec((2 * D, H), lambda i: (0, 0)),
            pl.BlockSpec((1, H), lambda i: (0, 0)),
            pl.BlockSpec((H, O), lambda i: (0, 0)),
            pl.BlockSpec((1, O), lambda i: (0, 0)),
        ],
        out_specs=pl.BlockSpec((O, _BM), lambda i: (0, i)),
        out_shape=jax.ShapeDtypeStruct((O, cb), jnp.float32),
    )

    b1r = b1.reshape(1, H)
    b2r = b2.reshape(1, O)
    W1bf = W1.astype(jnp.bfloat16)
    W2bf = W2.astype(jnp.bfloat16)
    outs = []
    for c in range(_NCHUNK):
        s = slice(c * cb, (c + 1) * cb)
        x1, x2 = gather(emb, in1[s], in2[s])
        outs.append(mlp(x1, x2, W1bf, b1r, W2bf, b2r))
    yt = outs[0] if _NCHUNK == 1 else jnp.concatenate(outs, axis=1)
    return yt.T


# speedup vs baseline: 1.0675x; 1.0675x over previous
import functools

import jax
import jax.numpy as jnp
from jax import lax
from jax.experimental import pallas as pl
from jax.experimental.pallas import tpu as pltpu
from jax.experimental.pallas import tpu_sc as plsc


def _make_sc_gather(V, D, B):
    info = plsc.get_sparse_core_info()
    nw = info.num_cores * info.num_subcores
    b_per_w = B // nw
    mesh = plsc.VectorSubcoreMesh(core_axis_name="c", subcore_axis_name="s")

    @functools.partial(
        pl.kernel,
        mesh=mesh,
        out_type=[
            jax.ShapeDtypeStruct((B, D), jnp.float32),
            jax.ShapeDtypeStruct((B, D), jnp.float32),
        ],
        scratch_types=[
            pltpu.VMEM((b_per_w,), jnp.int32),
            pltpu.VMEM((b_per_w, D), jnp.float32),
            pltpu.SemaphoreType.DMA,
        ],
    )
    def gather_k(emb_hbm, idx1_hbm, idx2_hbm, out1_hbm, out2_hbm,
                 idx_v, rows_v, sem):
        wid = lax.axis_index("s") * info.num_cores + lax.axis_index("c")
        base = wid * b_per_w
        pltpu.sync_copy(idx1_hbm.at[pl.ds(base, b_per_w)], idx_v)
        pltpu.async_copy(emb_hbm.at[idx_v], rows_v, sem).wait()
        pltpu.sync_copy(rows_v, out1_hbm.at[pl.ds(base, b_per_w)])
        pltpu.sync_copy(idx2_hbm.at[pl.ds(base, b_per_w)], idx_v)
        pltpu.async_copy(emb_hbm.at[idx_v], rows_v, sem).wait()
        pltpu.sync_copy(rows_v, out2_hbm.at[pl.ds(base, b_per_w)])

    return gather_k


_HC = 250


_SUB = 16


def _mlp_body(x1_ref, x2_ref, w1_ref, b1_ref, w2_ref, b2_ref, o_ref):
    d = x1_ref.shape[1]
    bm = x1_ref.shape[0]
    sb = bm // _SUB
    w1 = w1_ref[...]
    w2 = w2_ref[...]
    b1 = b1_ref[...]
    b2 = b2_ref[...]
    for j in range(_SUB):
        r = pl.ds(j * sb, sb)
        x = jnp.concatenate([x1_ref[r, :].astype(jnp.bfloat16),
                             x2_ref[r, :].astype(jnp.bfloat16)], axis=1)
        h = jnp.dot(x, w1, preferred_element_type=jnp.float32)
        h = jnp.maximum((h + b1).astype(jnp.bfloat16),
                        jnp.bfloat16(0.0))
        o = jnp.dot(h, w2, preferred_element_type=jnp.float32)
        e = jnp.exp(o + b2)
        y = e / jnp.sum(e, axis=1, keepdims=True)
        o_ref[:, r] = y.T


_NCHUNK = 1
_BM = 4096


def kernel(in1, in2, emb, W1, b1, W2, b2):
    B = in1.shape[0]
    V, D = emb.shape
    H = W1.shape[1]
    O = W2.shape[1]

    in1 = in1.astype(jnp.int32)
    in2 = in2.astype(jnp.int32)

    cb = B // _NCHUNK
    gather = _make_sc_gather(V, D, cb)

    mlp = pl.pallas_call(
        _mlp_body,
        grid=(cb // _BM,),
        in_specs=[
            pl.BlockSpec((_BM, D), lambda i: (i, 0)),
            pl.BlockSpec((_BM, D), lambda i: (i, 0)),
            pl.BlockSpec((2 * D, H), lambda i: (0, 0)),
            pl.BlockSpec((1, H), lambda i: (0, 0)),
            pl.BlockSpec((H, O), lambda i: (0, 0)),
            pl.BlockSpec((1, O), lambda i: (0, 0)),
        ],
        out_specs=pl.BlockSpec((O, _BM), lambda i: (0, i)),
        out_shape=jax.ShapeDtypeStruct((O, cb), jnp.float32),
    )

    b1r = b1.reshape(1, H)
    b2r = b2.reshape(1, O)
    W1bf = W1.astype(jnp.bfloat16)
    W2bf = W2.astype(jnp.bfloat16)
    outs = []
    for c in range(_NCHUNK):
        s = slice(c * cb, (c + 1) * cb)
        x1, x2 = gather(emb, in1[s], in2[s])
        outs.append(mlp(x1, x2, W1bf, b1r, W2bf, b2r))
    yt = outs[0] if _NCHUNK == 1 else jnp.concatenate(outs, axis=1)
    return yt.T


# speedup vs baseline: 1.0751x; 1.0071x over previous
import functools

import jax
import jax.numpy as jnp
from jax import lax
from jax.experimental import pallas as pl
from jax.experimental.pallas import tpu as pltpu
from jax.experimental.pallas import tpu_sc as plsc


def _make_sc_gather(V, D, B):
    info = plsc.get_sparse_core_info()
    nw = info.num_cores * info.num_subcores
    b_per_w = B // nw
    mesh = plsc.VectorSubcoreMesh(core_axis_name="c", subcore_axis_name="s")

    @functools.partial(
        pl.kernel,
        mesh=mesh,
        out_type=jax.ShapeDtypeStruct((B, 2 * D), jnp.float32),
        scratch_types=[
            pltpu.VMEM((b_per_w,), jnp.int32),
            pltpu.VMEM((b_per_w, D), jnp.float32),
            pltpu.SemaphoreType.DMA,
        ],
    )
    def gather_k(emb_hbm, idx1_hbm, idx2_hbm, out_hbm, idx_v, rows_v, sem):
        wid = lax.axis_index("s") * info.num_cores + lax.axis_index("c")
        base = wid * b_per_w
        pltpu.sync_copy(idx1_hbm.at[pl.ds(base, b_per_w)], idx_v)
        pltpu.async_copy(emb_hbm.at[idx_v], rows_v, sem).wait()
        pltpu.sync_copy(rows_v,
                        out_hbm.at[pl.ds(base, b_per_w), pl.ds(0, D)])
        pltpu.sync_copy(idx2_hbm.at[pl.ds(base, b_per_w)], idx_v)
        pltpu.async_copy(emb_hbm.at[idx_v], rows_v, sem).wait()
        pltpu.sync_copy(rows_v,
                        out_hbm.at[pl.ds(base, b_per_w), pl.ds(D, D)])

    return gather_k


_HC = 250


_SUB = 16


def _mlp_body(x_ref, w1_ref, b1_ref, w2_ref, b2_ref, o_ref):
    bm = x_ref.shape[0]
    sb = bm // _SUB
    w1 = w1_ref[...]
    w2 = w2_ref[...]
    b1 = b1_ref[...]
    b2 = b2_ref[...]
    for j in range(_SUB):
        r = pl.ds(j * sb, sb)
        x = x_ref[r, :].astype(jnp.bfloat16)
        h = jnp.dot(x, w1, preferred_element_type=jnp.float32)
        h = jnp.maximum((h + b1).astype(jnp.bfloat16),
                        jnp.bfloat16(0.0))
        o = jnp.dot(h, w2, preferred_element_type=jnp.float32)
        e = jnp.exp(o + b2)
        y = e / jnp.sum(e, axis=1, keepdims=True)
        o_ref[:, r] = y.T


_NCHUNK = 1
_BM = 4096


def kernel(in1, in2, emb, W1, b1, W2, b2):
    B = in1.shape[0]
    V, D = emb.shape
    H = W1.shape[1]
    O = W2.shape[1]

    in1 = in1.astype(jnp.int32)
    in2 = in2.astype(jnp.int32)

    cb = B // _NCHUNK
    gather = _make_sc_gather(V, D, cb)

    mlp = pl.pallas_call(
        _mlp_body,
        grid=(cb // _BM,),
        in_specs=[
            pl.BlockSpec((_BM, 2 * D), lambda i: (i, 0)),
            pl.BlockSpec((2 * D, H), lambda i: (0, 0)),
            pl.BlockSpec((1, H), lambda i: (0, 0)),
            pl.BlockSpec((H, O), lambda i: (0, 0)),
            pl.BlockSpec((1, O), lambda i: (0, 0)),
        ],
        out_specs=pl.BlockSpec((O, _BM), lambda i: (0, i)),
        out_shape=jax.ShapeDtypeStruct((O, cb), jnp.float32),
    )

    b1r = b1.reshape(1, H)
    b2r = b2.reshape(1, O)
    W1bf = W1.astype(jnp.bfloat16)
    W2bf = W2.astype(jnp.bfloat16)
    outs = []
    for c in range(_NCHUNK):
        s = slice(c * cb, (c + 1) * cb)
        x = gather(emb, in1[s], in2[s])
        outs.append(mlp(x, W1bf, b1r, W2bf, b2r))
    yt = outs[0] if _NCHUNK == 1 else jnp.concatenate(outs, axis=1)
    return yt.T
